# baseline (speedup 1.0000x reference)
import functools

import jax
import jax.numpy as jnp
from jax import lax
from jax.experimental import pallas as pl
from jax.experimental.pallas import tpu as pltpu
from jax.experimental.pallas import tpu_sc as plsc

_NC = 2
_NS = 16
_NW = _NC * _NS

_NB = 8


@functools.lru_cache(maxsize=None)
def _gather_kernel(B, T, V, D, nb):
    imgs_per_w = B // _NW
    n_per_w = imgs_per_w * T
    assert imgs_per_w % nb == 0
    mesh = plsc.VectorSubcoreMesh(core_axis_name="c", subcore_axis_name="s")

    @functools.partial(
        pl.kernel,
        mesh=mesh,
        compiler_params=pltpu.CompilerParams(use_tc_tiling_on_sc=False),
        out_type=jax.ShapeDtypeStruct((B, T, D), jnp.float32),
        scratch_types=[
            pltpu.VMEM((n_per_w,), jnp.int32),
            pltpu.VMEM((nb, T, D), jnp.float32),
        ]
        + [pltpu.SemaphoreType.DMA] * (2 * nb),
    )
    def k(x_hbm, w_hbm, out_hbm, idx_v, rows_v, *sems):
        gsem, wsem = sems[:nb], sems[nb:]
        wid = lax.axis_index("s") * _NC + lax.axis_index("c")
        base_img = wid * imgs_per_w
        pltpu.sync_copy(x_hbm.at[pl.ds(base_img * T, n_per_w)], idx_v)

        def gather(g, b):
            pltpu.async_copy(
                w_hbm.at[idx_v.at[pl.ds(g * T, T)]], rows_v.at[b], gsem[b]
            )

        for b in range(nb):
            gather(b, b)

        def body(i, carry):
            g0 = i * nb
            for b in range(nb):
                pltpu.make_async_copy(
                    w_hbm.at[idx_v.at[pl.ds(0, T)]], rows_v.at[b], gsem[b]
                ).wait()
                pltpu.async_copy(
                    rows_v.at[b], out_hbm.at[base_img + g0 + b], wsem[b]
                )
            for b in range(nb):
                pltpu.make_async_copy(
                    rows_v.at[b], out_hbm.at[base_img], wsem[b]
                ).wait()
                gather(jnp.minimum(g0 + nb + b, imgs_per_w - 1), b)
            return carry

        lax.fori_loop(0, imgs_per_w // nb, body, 0)

        for b in range(nb):
            pltpu.make_async_copy(
                w_hbm.at[idx_v.at[pl.ds(0, T)]], rows_v.at[b], gsem[b]
            ).wait()

    return k


def kernel(x, weight):
    B, T = x.shape
    V, D = weight.shape
    return _gather_kernel(B, T, V, D, _NB)(x.reshape(B * T), weight)


# speedup vs baseline: 1.0067x; 1.0067x over previous
import functools

import jax
import jax.numpy as jnp
from jax import lax
from jax.experimental import pallas as pl
from jax.experimental.pallas import tpu as pltpu
from jax.experimental.pallas import tpu_sc as plsc

_NC = 2
_NS = 16
_NW = _NC * _NS

_NB = 4


@functools.lru_cache(maxsize=None)
def _gather_kernel(B, T, V, D, nb):
    imgs_per_w = B // _NW
    n_per_w = imgs_per_w * T
    assert imgs_per_w % nb == 0
    mesh = plsc.VectorSubcoreMesh(core_axis_name="c", subcore_axis_name="s")

    @functools.partial(
        pl.kernel,
        mesh=mesh,
        compiler_params=pltpu.CompilerParams(use_tc_tiling_on_sc=False),
        out_type=jax.ShapeDtypeStruct((B, T, D), jnp.float32),
        scratch_types=[
            pltpu.VMEM((n_per_w,), jnp.int32),
            pltpu.VMEM((nb, T, D), jnp.float32),
        ]
        + [pltpu.SemaphoreType.DMA] * (2 * nb),
    )
    def k(x_hbm, w_hbm, out_hbm, idx_v, rows_v, *sems):
        gsem, wsem = sems[:nb], sems[nb:]
        wid = lax.axis_index("s") * _NC + lax.axis_index("c")
        base_img = wid * imgs_per_w
        pltpu.sync_copy(x_hbm.at[pl.ds(base_img * T, n_per_w)], idx_v)

        def gather(g, b):
            pltpu.async_copy(
                w_hbm.at[idx_v.at[pl.ds(g * T, T)]], rows_v.at[b], gsem[b]
            )

        for b in range(nb):
            gather(b, b)

        def body(i, carry):
            g0 = i * nb
            for b in range(nb):
                pltpu.make_async_copy(
                    w_hbm.at[idx_v.at[pl.ds(0, T)]], rows_v.at[b], gsem[b]
                ).wait()
                pltpu.async_copy(
                    rows_v.at[b], out_hbm.at[base_img + g0 + b], wsem[b]
                )
            for b in range(nb):
                pltpu.make_async_copy(
                    rows_v.at[b], out_hbm.at[base_img], wsem[b]
                ).wait()
                gather(jnp.minimum(g0 + nb + b, imgs_per_w - 1), b)
            return carry

        lax.fori_loop(0, imgs_per_w // nb, body, 0)

        for b in range(nb):
            pltpu.make_async_copy(
                w_hbm.at[idx_v.at[pl.ds(0, T)]], rows_v.at[b], gsem[b]
            ).wait()

    return k


def kernel(x, weight):
    B, T = x.shape
    V, D = weight.shape
    return _gather_kernel(B, T, V, D, _NB)(x.reshape(B * T), weight)
